# baseline (speedup 1.0000x reference)
import functools
import math

import jax
import jax.numpy as jnp
from jax import lax
from jax.experimental import pallas as pl
from jax.experimental.pallas import tpu as pltpu
from jax.experimental.pallas import tpu_sc as plsc

_NC = 2
_NS = 16
_NW = _NC * _NS

_TOPK = 2
_CF = 1.25
_TB = 256


def _router_body(C, E, TB, x_ref, wr_ref, br_ref, d_ref, c_ref, w_ref,
                 carry_ref):
    i = pl.program_id(0)

    @pl.when(i == 0)
    def _():
        carry_ref[...] = jnp.zeros_like(carry_ref)

    xb = x_ref[...]
    logits = (jnp.dot(xb, wr_ref[...], preferred_element_type=jnp.float32,
                      precision=lax.Precision.DEFAULT) + br_ref[...])
    m = jnp.max(logits, axis=1, keepdims=True)
    ex = jnp.exp(logits - m)
    probs = ex / jnp.sum(ex, axis=1, keepdims=True)

    col = lax.broadcasted_iota(jnp.int32, (TB, E), 1)
    p1 = jnp.max(probs, axis=1)
    i1 = jnp.min(jnp.where(probs == p1[:, None], col, E), axis=1)
    masked = jnp.where(col == i1[:, None], -1e30, probs)
    p2 = jnp.max(masked, axis=1)
    i2 = jnp.min(jnp.where(masked == p2[:, None], col, E), axis=1)
    s = jnp.maximum(p1 + p2, 1e-9)
    w1 = p1 / s
    w2 = p2 / s

    oh1 = (col == i1[:, None]).astype(jnp.float32)
    oh2 = (col == i2[:, None]).astype(jnp.float32)
    cnt = oh1 + oh2
    r = lax.broadcasted_iota(jnp.int32, (TB, TB), 0)
    c = lax.broadcasted_iota(jnp.int32, (TB, TB), 1)
    tri = (c < r).astype(jnp.float32)
    ex_in = jnp.dot(tri, cnt, preferred_element_type=jnp.float32,
                    precision=lax.Precision.HIGHEST)
    tot = ex_in + carry_ref[0:1, :]
    carry_ref[0:1, :] = carry_ref[0:1, :] + jnp.sum(cnt, axis=0,
                                                    keepdims=True)

    rank1 = jnp.sum(tot * oh1, axis=1).astype(jnp.int32)
    rank2 = jnp.sum(tot * oh2, axis=1).astype(jnp.int32)
    kept1 = rank1 < C
    kept2 = rank2 < C
    base1 = i1 * C + rank1
    base2 = i2 * C + rank2

    rowid = lax.broadcasted_iota(jnp.int32, (TB,), 0)
    wid = (i * TB + rowid) // (TB // 2)
    trash1 = E * C + 2 * (wid % _NW)
    trash2 = trash1 + 1
    d_ref[0, :] = jnp.where(kept1, base1, trash1)
    d_ref[1, :] = jnp.where(kept2, base2, trash2)
    c_ref[0, :] = jnp.where(kept1, base1, 0)
    c_ref[1, :] = jnp.where(kept2, base2, 0)
    w_ref[0, :] = jnp.where(kept1, w1, 0.0)
    w_ref[1, :] = jnp.where(kept2, w2, 0.0)


def _run_router(xf, Wr, br, C):
    T, D = xf.shape
    E = Wr.shape[1]
    n = T // _TB
    out1 = jax.ShapeDtypeStruct((2, T), jnp.int32)
    outf = jax.ShapeDtypeStruct((2, T), jnp.float32)
    vec_spec = pl.BlockSpec((2, _TB), lambda i: (0, i))
    return pl.pallas_call(
        functools.partial(_router_body, C, E, _TB),
        grid=(n,),
        in_specs=[
            pl.BlockSpec((_TB, D), lambda i: (i, 0)),
            pl.BlockSpec((D, E), lambda i: (0, 0)),
            pl.BlockSpec((1, E), lambda i: (0, 0)),
        ],
        out_specs=[vec_spec] * 3,
        out_shape=[out1, out1, outf],
        scratch_shapes=[pltpu.VMEM((8, E), jnp.float32)],
        compiler_params=pltpu.CompilerParams(
            dimension_semantics=("arbitrary",)),
    )(xf, Wr, br.reshape(1, E))


def _run_dispatch(xf, dst, rows_out):
    T, D = xf.shape
    tpw = T // _NW
    mesh = plsc.VectorSubcoreMesh(core_axis_name="c", subcore_axis_name="s")

    @functools.partial(
        pl.kernel,
        out_type=jax.ShapeDtypeStruct((rows_out, D), jnp.float32),
        mesh=mesh,
        scratch_types=[
            pltpu.VMEM((tpw, D), jnp.float32),
            pltpu.VMEM((2, tpw), jnp.int32),
            pltpu.SemaphoreType.DMA,
            pltpu.SemaphoreType.DMA,
        ],
    )
    def disp(xf_hbm, dst_hbm, xe_hbm, rows_v, idx_v, sem1, sem2):
        wid = lax.axis_index("s") * _NC + lax.axis_index("c")
        base = wid * tpw
        pltpu.sync_copy(xf_hbm.at[pl.ds(base, tpw)], rows_v)
        pltpu.sync_copy(dst_hbm.at[:, pl.ds(base, tpw)], idx_v)
        cp1 = pltpu.async_copy(rows_v, xe_hbm.at[idx_v.at[0]], sem1)
        cp2 = pltpu.async_copy(rows_v, xe_hbm.at[idx_v.at[1]], sem2)
        cp1.wait()
        cp2.wait()

    return disp(xf, dst)


def _ffn_body(x_ref, w1_ref, b1_ref, w2_ref, b2_ref, o_ref):
    xb = x_ref[...]
    h = (jnp.dot(xb, w1_ref[0], preferred_element_type=jnp.float32,
                 precision=lax.Precision.DEFAULT) + b1_ref[0])
    h = 0.5 * h * (1.0 + lax.erf(h * (1.0 / math.sqrt(2.0))))
    o_ref[0] = (jnp.dot(h, w2_ref[0], preferred_element_type=jnp.float32,
                        precision=lax.Precision.DEFAULT) + b2_ref[0])


def _run_ffn(xe, W1, b1, W2, b2, C):
    E, D, F = W1.shape
    return pl.pallas_call(
        _ffn_body,
        grid=(E,),
        in_specs=[
            pl.BlockSpec((C, D), lambda e: (e, 0)),
            pl.BlockSpec((1, D, F), lambda e: (e, 0, 0)),
            pl.BlockSpec((1, 1, F), lambda e: (e, 0, 0)),
            pl.BlockSpec((1, F, D), lambda e: (e, 0, 0)),
            pl.BlockSpec((1, 1, D), lambda e: (e, 0, 0)),
        ],
        out_specs=pl.BlockSpec((1, C, D), lambda e: (e, 0, 0)),
        out_shape=jax.ShapeDtypeStruct((E, C, D), jnp.float32),
        compiler_params=pltpu.CompilerParams(
            dimension_semantics=("arbitrary",)),
    )(xe, W1, b1.reshape(E, 1, F), W2, b2.reshape(E, 1, D))


def _run_combine(ye, cidx, cw, T, D):
    tpw = T // _NW
    half = tpw // 2
    nvec = D // 16
    mesh = plsc.VectorSubcoreMesh(core_axis_name="c", subcore_axis_name="s")

    @functools.partial(
        pl.kernel,
        out_type=jax.ShapeDtypeStruct((T, D), jnp.float32),
        mesh=mesh,
        scratch_types=[
            pltpu.VMEM((half, D), jnp.float32),
            pltpu.VMEM((half, D), jnp.float32),
            pltpu.VMEM((2, tpw), jnp.int32),
            pltpu.VMEM((tpw,), jnp.float32),
            pltpu.VMEM((tpw,), jnp.float32),
            pltpu.SemaphoreType.DMA,
            pltpu.SemaphoreType.DMA,
        ],
        compiler_params=pltpu.CompilerParams(needs_layout_passes=False),
    )
    def comb(ye_hbm, ci_hbm, cw_hbm, out_hbm, g1_v, g2_v, idx_v, w1_v,
             w2_v, sem1, sem2):
        wid = lax.axis_index("s") * _NC + lax.axis_index("c")
        base = wid * tpw
        pltpu.sync_copy(ci_hbm.at[:, pl.ds(base, tpw)], idx_v)
        pltpu.sync_copy(cw_hbm.at[0, pl.ds(base, tpw)], w1_v)
        pltpu.sync_copy(cw_hbm.at[1, pl.ds(base, tpw)], w2_v)
        for h in range(2):
            cp1 = pltpu.async_copy(
                ye_hbm.at[idx_v.at[0, pl.ds(h * half, half)]], g1_v, sem1)
            cp2 = pltpu.async_copy(
                ye_hbm.at[idx_v.at[1, pl.ds(h * half, half)]], g2_v, sem2)
            cp1.wait()
            cp2.wait()

            def row(r, _, h=h):
                ridx = jnp.full((16,), h * half + r, jnp.int32)
                w1s = plsc.load_gather(w1_v, [ridx])
                w2s = plsc.load_gather(w2_v, [ridx])
                zero = jnp.zeros((16,), jnp.float32)
                for j in range(nvec):
                    a = g1_v[r, pl.ds(j * 16, 16)]
                    b = g2_v[r, pl.ds(j * 16, 16)]
                    va = jnp.where(w1s != 0.0, a * w1s, zero)
                    vb = jnp.where(w2s != 0.0, b * w2s, zero)
                    g1_v[r, pl.ds(j * 16, 16)] = va + vb
                return 0

            lax.fori_loop(0, half, row, 0)
            pltpu.sync_copy(g1_v, out_hbm.at[pl.ds(base + h * half, half)])

    return comb(ye, cidx, cw)


def kernel(x, Wr, br, W1, b1, W2, b2):
    B, S, D = x.shape
    T = B * S
    E = Wr.shape[1]
    F = W1.shape[2]
    C = max(1, int(math.ceil(_CF * T * _TOPK / E)))
    EC = E * C
    rows_out = EC + 2 * _NW

    xf = x.reshape(T, D)
    dst, cidx, cw = _run_router(xf, Wr, br, C)
    xe = _run_dispatch(xf, dst, rows_out)
    ye = _run_ffn(xe, W1, b1, W2, b2, C)
    out = _run_combine(ye.reshape(EC, D), cidx, cw, T, D)
    return out.reshape(B, S, D)


# speedup vs baseline: 1.0870x; 1.0870x over previous
import functools
import math

import jax
import jax.numpy as jnp
from jax import lax
from jax.experimental import pallas as pl
from jax.experimental.pallas import tpu as pltpu
from jax.experimental.pallas import tpu_sc as plsc

_NC = 2
_NS = 16
_NW = _NC * _NS

_TOPK = 2
_CF = 1.25
_TB = 256


def _router_body(C, E, TB, x_ref, wr_ref, br_ref, d_ref, c_ref, w_ref,
                 carry_ref):
    i = pl.program_id(0)

    @pl.when(i == 0)
    def _():
        carry_ref[...] = jnp.zeros_like(carry_ref)

    xb = x_ref[...]
    logits = (jnp.dot(xb, wr_ref[...], preferred_element_type=jnp.float32,
                      precision=lax.Precision.DEFAULT) + br_ref[...])
    lt = logits.T
    m = jnp.max(lt, axis=0, keepdims=True)
    ex = jnp.exp(lt - m)
    probs = ex / jnp.sum(ex, axis=0, keepdims=True)

    col = lax.broadcasted_iota(jnp.int32, (E, TB), 0)
    p1 = jnp.max(probs, axis=0, keepdims=True)
    i1 = jnp.min(jnp.where(probs == p1, col, E), axis=0, keepdims=True)
    masked = jnp.where(col == i1, -1e30, probs)
    p2 = jnp.max(masked, axis=0, keepdims=True)
    i2 = jnp.min(jnp.where(masked == p2, col, E), axis=0, keepdims=True)
    s = jnp.maximum(p1 + p2, 1e-9)
    w1 = p1 / s
    w2 = p2 / s

    oh1 = (col == i1).astype(jnp.float32)
    oh2 = (col == i2).astype(jnp.float32)
    cnt = oh1 + oh2
    r = lax.broadcasted_iota(jnp.int32, (TB, TB), 0)
    c = lax.broadcasted_iota(jnp.int32, (TB, TB), 1)
    tri = (r < c).astype(jnp.float32)
    ex_in = jnp.dot(cnt, tri, preferred_element_type=jnp.float32,
                    precision=lax.Precision.HIGHEST)
    tot = ex_in + carry_ref[:, 0:1]
    carry_ref[:, 0:1] = carry_ref[:, 0:1] + jnp.sum(cnt, axis=1,
                                                    keepdims=True)

    rank1 = jnp.sum(tot * oh1, axis=0, keepdims=True).astype(jnp.int32)
    rank2 = jnp.sum(tot * oh2, axis=0, keepdims=True).astype(jnp.int32)
    kept1 = rank1 < C
    kept2 = rank2 < C
    base1 = i1 * C + rank1
    base2 = i2 * C + rank2

    rowid = lax.broadcasted_iota(jnp.int32, (1, TB), 1)
    wid = (i * TB + rowid) // (TB // 2)
    trash1 = E * C + 2 * (wid % _NW)
    trash2 = trash1 + 1
    d_ref[0:1, :] = jnp.where(kept1, base1, trash1)
    d_ref[1:2, :] = jnp.where(kept2, base2, trash2)
    c_ref[0:1, :] = jnp.where(kept1, base1, 0)
    c_ref[1:2, :] = jnp.where(kept2, base2, 0)
    w_ref[0:1, :] = jnp.where(kept1, w1, 0.0)
    w_ref[1:2, :] = jnp.where(kept2, w2, 0.0)


def _run_router(xf, Wr, br, C):
    T, D = xf.shape
    E = Wr.shape[1]
    n = T // _TB
    out1 = jax.ShapeDtypeStruct((2, T), jnp.int32)
    outf = jax.ShapeDtypeStruct((2, T), jnp.float32)
    vec_spec = pl.BlockSpec((2, _TB), lambda i: (0, i))
    return pl.pallas_call(
        functools.partial(_router_body, C, E, _TB),
        grid=(n,),
        in_specs=[
            pl.BlockSpec((_TB, D), lambda i: (i, 0)),
            pl.BlockSpec((D, E), lambda i: (0, 0)),
            pl.BlockSpec((1, E), lambda i: (0, 0)),
        ],
        out_specs=[vec_spec] * 3,
        out_shape=[out1, out1, outf],
        scratch_shapes=[pltpu.VMEM((E, 128), jnp.float32)],
        compiler_params=pltpu.CompilerParams(
            dimension_semantics=("arbitrary",)),
    )(xf, Wr, br.reshape(1, E))


def _run_dispatch(xf, dst, rows_out):
    T, D = xf.shape
    tpw = T // _NW
    mesh = plsc.VectorSubcoreMesh(core_axis_name="c", subcore_axis_name="s")

    @functools.partial(
        pl.kernel,
        out_type=jax.ShapeDtypeStruct((rows_out, D), jnp.float32),
        mesh=mesh,
        scratch_types=[
            pltpu.VMEM((tpw, D), jnp.float32),
            pltpu.VMEM((2, tpw), jnp.int32),
            pltpu.SemaphoreType.DMA,
            pltpu.SemaphoreType.DMA,
        ],
    )
    def disp(xf_hbm, dst_hbm, xe_hbm, rows_v, idx_v, sem1, sem2):
        wid = lax.axis_index("s") * _NC + lax.axis_index("c")
        base = wid * tpw
        pltpu.sync_copy(xf_hbm.at[pl.ds(base, tpw)], rows_v)
        pltpu.sync_copy(dst_hbm.at[:, pl.ds(base, tpw)], idx_v)
        cp1 = pltpu.async_copy(rows_v, xe_hbm.at[idx_v.at[0]], sem1)
        cp2 = pltpu.async_copy(rows_v, xe_hbm.at[idx_v.at[1]], sem2)
        cp1.wait()
        cp2.wait()

    return disp(xf, dst)


def _ffn_body(x_ref, w1_ref, b1_ref, w2_ref, b2_ref, o_ref):
    xb = x_ref[...]
    h = (jnp.dot(xb, w1_ref[0], preferred_element_type=jnp.float32,
                 precision=lax.Precision.DEFAULT) + b1_ref[0])
    h = 0.5 * h * (1.0 + lax.erf(h * (1.0 / math.sqrt(2.0))))
    o_ref[0] = (jnp.dot(h, w2_ref[0], preferred_element_type=jnp.float32,
                        precision=lax.Precision.DEFAULT) + b2_ref[0])


def _run_ffn(xe, W1, b1, W2, b2, C):
    E, D, F = W1.shape
    return pl.pallas_call(
        _ffn_body,
        grid=(E,),
        in_specs=[
            pl.BlockSpec((C, D), lambda e: (e, 0)),
            pl.BlockSpec((1, D, F), lambda e: (e, 0, 0)),
            pl.BlockSpec((1, 1, F), lambda e: (e, 0, 0)),
            pl.BlockSpec((1, F, D), lambda e: (e, 0, 0)),
            pl.BlockSpec((1, 1, D), lambda e: (e, 0, 0)),
        ],
        out_specs=pl.BlockSpec((1, C, D), lambda e: (e, 0, 0)),
        out_shape=jax.ShapeDtypeStruct((E, C, D), jnp.float32),
        compiler_params=pltpu.CompilerParams(
            dimension_semantics=("arbitrary",)),
    )(xe, W1, b1.reshape(E, 1, F), W2, b2.reshape(E, 1, D))


def _run_combine(ye, cidx, cw, T, D):
    tpw = T // _NW
    half = tpw // 2
    nvec = D // 16
    mesh = plsc.VectorSubcoreMesh(core_axis_name="c", subcore_axis_name="s")

    @functools.partial(
        pl.kernel,
        out_type=jax.ShapeDtypeStruct((T, D), jnp.float32),
        mesh=mesh,
        scratch_types=[
            pltpu.VMEM((half, D), jnp.float32),
            pltpu.VMEM((half, D), jnp.float32),
            pltpu.VMEM((2, tpw), jnp.int32),
            pltpu.VMEM((tpw,), jnp.float32),
            pltpu.VMEM((tpw,), jnp.float32),
            pltpu.SemaphoreType.DMA,
            pltpu.SemaphoreType.DMA,
        ],
        compiler_params=pltpu.CompilerParams(needs_layout_passes=False),
    )
    def comb(ye_hbm, ci_hbm, cw_hbm, out_hbm, g1_v, g2_v, idx_v, w1_v,
             w2_v, sem1, sem2):
        wid = lax.axis_index("s") * _NC + lax.axis_index("c")
        base = wid * tpw
        pltpu.sync_copy(ci_hbm.at[:, pl.ds(base, tpw)], idx_v)
        pltpu.sync_copy(cw_hbm.at[0, pl.ds(base, tpw)], w1_v)
        pltpu.sync_copy(cw_hbm.at[1, pl.ds(base, tpw)], w2_v)
        for h in range(2):
            cp1 = pltpu.async_copy(
                ye_hbm.at[idx_v.at[0, pl.ds(h * half, half)]], g1_v, sem1)
            cp2 = pltpu.async_copy(
                ye_hbm.at[idx_v.at[1, pl.ds(h * half, half)]], g2_v, sem2)
            cp1.wait()
            cp2.wait()

            def row(r, _, h=h):
                ridx = jnp.full((16,), h * half + r, jnp.int32)
                w1s = plsc.load_gather(w1_v, [ridx])
                w2s = plsc.load_gather(w2_v, [ridx])
                zero = jnp.zeros((16,), jnp.float32)
                for j in range(nvec):
                    a = g1_v[r, pl.ds(j * 16, 16)]
                    b = g2_v[r, pl.ds(j * 16, 16)]
                    va = jnp.where(w1s != 0.0, a * w1s, zero)
                    vb = jnp.where(w2s != 0.0, b * w2s, zero)
                    g1_v[r, pl.ds(j * 16, 16)] = va + vb
                return 0

            lax.fori_loop(0, half, row, 0)
            pltpu.sync_copy(g1_v, out_hbm.at[pl.ds(base + h * half, half)])

    return comb(ye, cidx, cw)


def kernel(x, Wr, br, W1, b1, W2, b2):
    B, S, D = x.shape
    T = B * S
    E = Wr.shape[1]
    F = W1.shape[2]
    C = max(1, int(math.ceil(_CF * T * _TOPK / E)))
    EC = E * C
    rows_out = EC + 2 * _NW

    xf = x.reshape(T, D)
    dst, cidx, cw = _run_router(xf, Wr, br, C)
    xe = _run_dispatch(xf, dst, rows_out)
    ye = _run_ffn(xe, W1, b1, W2, b2, C)
    out = _run_combine(ye.reshape(EC, D), cidx, cw, T, D)
    return out.reshape(B, S, D)


# speedup vs baseline: 1.0953x; 1.0077x over previous
import functools
import math

import jax
import jax.numpy as jnp
from jax import lax
from jax.experimental import pallas as pl
from jax.experimental.pallas import tpu as pltpu
from jax.experimental.pallas import tpu_sc as plsc

_NC = 2
_NS = 16
_NW = _NC * _NS

_TOPK = 2
_CF = 1.25
_TB = 256


def _router_body(C, E, TB, x_ref, wr_ref, br_ref, d_ref, c_ref, w_ref,
                 carry_ref):
    i = pl.program_id(0)

    @pl.when(i == 0)
    def _():
        carry_ref[...] = jnp.zeros_like(carry_ref)

    xb = x_ref[...]
    logits = (jnp.dot(xb, wr_ref[...], preferred_element_type=jnp.float32,
                      precision=lax.Precision.DEFAULT) + br_ref[...])
    lt = logits.T
    m = jnp.max(lt, axis=0, keepdims=True)
    ex = jnp.exp(lt - m)
    probs = ex / jnp.sum(ex, axis=0, keepdims=True)

    col = lax.broadcasted_iota(jnp.int32, (E, TB), 0)
    p1 = jnp.max(probs, axis=0, keepdims=True)
    i1 = jnp.min(jnp.where(probs == p1, col, E), axis=0, keepdims=True)
    masked = jnp.where(col == i1, -1e30, probs)
    p2 = jnp.max(masked, axis=0, keepdims=True)
    i2 = jnp.min(jnp.where(masked == p2, col, E), axis=0, keepdims=True)
    s = jnp.maximum(p1 + p2, 1e-9)
    w1 = p1 / s
    w2 = p2 / s

    oh1 = (col == i1).astype(jnp.float32)
    oh2 = (col == i2).astype(jnp.float32)
    cnt = oh1 + oh2
    r = lax.broadcasted_iota(jnp.int32, (TB, TB), 0)
    c = lax.broadcasted_iota(jnp.int32, (TB, TB), 1)
    tri = (r < c).astype(jnp.float32)
    ex_in = jnp.dot(cnt, tri, preferred_element_type=jnp.float32,
                    precision=lax.Precision.HIGHEST)
    tot = ex_in + carry_ref[:, 0:1]
    carry_ref[:, 0:1] = carry_ref[:, 0:1] + jnp.sum(cnt, axis=1,
                                                    keepdims=True)

    rank1 = jnp.sum(tot * oh1, axis=0, keepdims=True).astype(jnp.int32)
    rank2 = jnp.sum(tot * oh2, axis=0, keepdims=True).astype(jnp.int32)
    kept1 = rank1 < C
    kept2 = rank2 < C
    base1 = i1 * C + rank1
    base2 = i2 * C + rank2

    rowid = lax.broadcasted_iota(jnp.int32, (1, TB), 1)
    wid = (i * TB + rowid) // (TB // 2)
    trash1 = E * C + 2 * (wid % _NW)
    trash2 = trash1 + 1
    d_ref[0:1, :] = jnp.where(kept1, base1, trash1)
    d_ref[1:2, :] = jnp.where(kept2, base2, trash2)
    c_ref[0:1, :] = jnp.where(kept1, base1, 0)
    c_ref[1:2, :] = jnp.where(kept2, base2, 0)
    w_ref[0:1, :] = jnp.where(kept1, w1, 0.0)
    w_ref[1:2, :] = jnp.where(kept2, w2, 0.0)


def _run_router(xf, Wr, br, C):
    T, D = xf.shape
    E = Wr.shape[1]
    n = T // _TB
    out1 = jax.ShapeDtypeStruct((2, T), jnp.int32)
    outf = jax.ShapeDtypeStruct((2, T), jnp.float32)
    vec_spec = pl.BlockSpec((2, _TB), lambda i: (0, i))
    return pl.pallas_call(
        functools.partial(_router_body, C, E, _TB),
        grid=(n,),
        in_specs=[
            pl.BlockSpec((_TB, D), lambda i: (i, 0)),
            pl.BlockSpec((D, E), lambda i: (0, 0)),
            pl.BlockSpec((1, E), lambda i: (0, 0)),
        ],
        out_specs=[vec_spec] * 3,
        out_shape=[out1, out1, outf],
        scratch_shapes=[pltpu.VMEM((E, 128), jnp.float32)],
        compiler_params=pltpu.CompilerParams(
            dimension_semantics=("arbitrary",)),
    )(xf, Wr, br.reshape(1, E))


def _run_dispatch(xf, dst, rows_out):
    T, D = xf.shape
    tpw = T // _NW
    mesh = plsc.VectorSubcoreMesh(core_axis_name="c", subcore_axis_name="s")

    @functools.partial(
        pl.kernel,
        out_type=jax.ShapeDtypeStruct((rows_out, D), jnp.float32),
        mesh=mesh,
        scratch_types=[
            pltpu.VMEM((tpw, D), jnp.float32),
            pltpu.VMEM((2, tpw), jnp.int32),
            pltpu.SemaphoreType.DMA,
            pltpu.SemaphoreType.DMA,
        ],
    )
    def disp(xf_hbm, dst_hbm, xe_hbm, rows_v, idx_v, sem1, sem2):
        wid = lax.axis_index("s") * _NC + lax.axis_index("c")
        base = wid * tpw
        pltpu.sync_copy(xf_hbm.at[pl.ds(base, tpw)], rows_v)
        pltpu.sync_copy(dst_hbm.at[:, pl.ds(base, tpw)], idx_v)
        cp1 = pltpu.async_copy(rows_v, xe_hbm.at[idx_v.at[0]], sem1)
        cp2 = pltpu.async_copy(rows_v, xe_hbm.at[idx_v.at[1]], sem2)
        cp1.wait()
        cp2.wait()

    return disp(xf, dst)


_EPB = 2


def _ffn_body(C, x_ref, w1_ref, b1_ref, w2_ref, b2_ref, o_ref):
    for j in range(_EPB):
        xb = x_ref[pl.ds(j * C, C), :]
        h = (jnp.dot(xb, w1_ref[j], preferred_element_type=jnp.float32,
                     precision=lax.Precision.DEFAULT) + b1_ref[j])
        h = 0.5 * h * (1.0 + lax.erf(h * (1.0 / math.sqrt(2.0))))
        o_ref[j] = (jnp.dot(h, w2_ref[j], preferred_element_type=jnp.float32,
                            precision=lax.Precision.DEFAULT) + b2_ref[j])


def _run_ffn(xe, W1, b1, W2, b2, C):
    E, D, F = W1.shape
    return pl.pallas_call(
        functools.partial(_ffn_body, C),
        grid=(E // _EPB,),
        in_specs=[
            pl.BlockSpec((_EPB * C, D), lambda e: (e, 0)),
            pl.BlockSpec((_EPB, D, F), lambda e: (e, 0, 0)),
            pl.BlockSpec((_EPB, 1, F), lambda e: (e, 0, 0)),
            pl.BlockSpec((_EPB, F, D), lambda e: (e, 0, 0)),
            pl.BlockSpec((_EPB, 1, D), lambda e: (e, 0, 0)),
        ],
        out_specs=pl.BlockSpec((_EPB, C, D), lambda e: (e, 0, 0)),
        out_shape=jax.ShapeDtypeStruct((E, C, D), jnp.float32),
        compiler_params=pltpu.CompilerParams(
            dimension_semantics=("arbitrary",)),
    )(xe, W1, b1.reshape(E, 1, F), W2, b2.reshape(E, 1, D))


def _run_combine(ye, cidx, cw, T, D):
    tpw = T // _NW
    half = tpw // 2
    nvec = D // 16
    mesh = plsc.VectorSubcoreMesh(core_axis_name="c", subcore_axis_name="s")

    @functools.partial(
        pl.kernel,
        out_type=jax.ShapeDtypeStruct((T, D), jnp.float32),
        mesh=mesh,
        scratch_types=[
            pltpu.VMEM((half, D), jnp.float32),
            pltpu.VMEM((half, D), jnp.float32),
            pltpu.VMEM((2, tpw), jnp.int32),
            pltpu.VMEM((tpw,), jnp.float32),
            pltpu.VMEM((tpw,), jnp.float32),
            pltpu.SemaphoreType.DMA,
            pltpu.SemaphoreType.DMA,
        ],
        compiler_params=pltpu.CompilerParams(needs_layout_passes=False),
    )
    def comb(ye_hbm, ci_hbm, cw_hbm, out_hbm, g1_v, g2_v, idx_v, w1_v,
             w2_v, sem1, sem2):
        wid = lax.axis_index("s") * _NC + lax.axis_index("c")
        base = wid * tpw
        pltpu.sync_copy(ci_hbm.at[:, pl.ds(base, tpw)], idx_v)
        pltpu.sync_copy(cw_hbm.at[0, pl.ds(base, tpw)], w1_v)
        pltpu.sync_copy(cw_hbm.at[1, pl.ds(base, tpw)], w2_v)
        for h in range(2):
            cp1 = pltpu.async_copy(
                ye_hbm.at[idx_v.at[0, pl.ds(h * half, half)]], g1_v, sem1)
            cp2 = pltpu.async_copy(
                ye_hbm.at[idx_v.at[1, pl.ds(h * half, half)]], g2_v, sem2)
            cp1.wait()
            cp2.wait()

            def row(r, _, h=h):
                ridx = jnp.full((16,), h * half + r, jnp.int32)
                w1s = plsc.load_gather(w1_v, [ridx])
                w2s = plsc.load_gather(w2_v, [ridx])
                zero = jnp.zeros((16,), jnp.float32)
                for j in range(nvec):
                    a = g1_v[r, pl.ds(j * 16, 16)]
                    b = g2_v[r, pl.ds(j * 16, 16)]
                    va = jnp.where(w1s != 0.0, a * w1s, zero)
                    vb = jnp.where(w2s != 0.0, b * w2s, zero)
                    g1_v[r, pl.ds(j * 16, 16)] = va + vb
                return 0

            lax.fori_loop(0, half, row, 0)
            pltpu.sync_copy(g1_v, out_hbm.at[pl.ds(base + h * half, half)])

    return comb(ye, cidx, cw)


def kernel(x, Wr, br, W1, b1, W2, b2):
    B, S, D = x.shape
    T = B * S
    E = Wr.shape[1]
    F = W1.shape[2]
    C = max(1, int(math.ceil(_CF * T * _TOPK / E)))
    EC = E * C
    rows_out = EC + 2 * _NW

    xf = x.reshape(T, D)
    dst, cidx, cw = _run_router(xf, Wr, br, C)
    xe = _run_dispatch(xf, dst, rows_out)
    ye = _run_ffn(xe, W1, b1, W2, b2, C)
    out = _run_combine(ye.reshape(EC, D), cidx, cw, T, D)
    return out.reshape(B, S, D)


# speedup vs baseline: 1.1152x; 1.0181x over previous
import functools
import math

import jax
import jax.numpy as jnp
from jax import lax
from jax.experimental import pallas as pl
from jax.experimental.pallas import tpu as pltpu
from jax.experimental.pallas import tpu_sc as plsc

_NC = 2
_NS = 16
_NW = _NC * _NS

_TOPK = 2
_CF = 1.25
_TB = 256


def _router_body(C, E, TB, x_ref, wr_ref, br_ref, d_ref, c_ref, w_ref,
                 carry_ref):
    i = pl.program_id(0)

    @pl.when(i == 0)
    def _():
        carry_ref[...] = jnp.zeros_like(carry_ref)

    xb = x_ref[...]
    logits = (jnp.dot(xb, wr_ref[...], preferred_element_type=jnp.float32,
                      precision=lax.Precision.DEFAULT) + br_ref[...])
    lt = logits.T
    m = jnp.max(lt, axis=0, keepdims=True)
    ex = jnp.exp(lt - m)
    probs = ex / jnp.sum(ex, axis=0, keepdims=True)

    col = lax.broadcasted_iota(jnp.int32, (E, TB), 0)
    p1 = jnp.max(probs, axis=0, keepdims=True)
    i1 = jnp.min(jnp.where(probs == p1, col, E), axis=0, keepdims=True)
    masked = jnp.where(col == i1, -1e30, probs)
    p2 = jnp.max(masked, axis=0, keepdims=True)
    i2 = jnp.min(jnp.where(masked == p2, col, E), axis=0, keepdims=True)
    s = jnp.maximum(p1 + p2, 1e-9)
    w1 = p1 / s
    w2 = p2 / s

    oh1 = (col == i1).astype(jnp.float32)
    oh2 = (col == i2).astype(jnp.float32)
    cnt = oh1 + oh2
    r = lax.broadcasted_iota(jnp.int32, (TB, TB), 0)
    c = lax.broadcasted_iota(jnp.int32, (TB, TB), 1)
    tri = (r < c).astype(jnp.float32)
    ex_in = jnp.dot(cnt, tri, preferred_element_type=jnp.float32,
                    precision=lax.Precision.HIGHEST)
    tot = ex_in + carry_ref[:, 0:1]
    carry_ref[:, 0:1] = carry_ref[:, 0:1] + jnp.sum(cnt, axis=1,
                                                    keepdims=True)

    rank1 = jnp.sum(tot * oh1, axis=0, keepdims=True).astype(jnp.int32)
    rank2 = jnp.sum(tot * oh2, axis=0, keepdims=True).astype(jnp.int32)
    kept1 = rank1 < C
    kept2 = rank2 < C
    base1 = i1 * C + rank1
    base2 = i2 * C + rank2

    rowid = lax.broadcasted_iota(jnp.int32, (1, TB), 1)
    wid = (i * TB + rowid) // (TB // 2)
    trash1 = E * C + 2 * (wid % _NW)
    trash2 = trash1 + 1
    d_ref[0:1, :] = jnp.where(kept1, base1, trash1)
    d_ref[1:2, :] = jnp.where(kept2, base2, trash2)
    c_ref[0:1, :] = jnp.where(kept1, base1, 0)
    c_ref[1:2, :] = jnp.where(kept2, base2, 0)
    w_ref[0:1, :] = jnp.where(kept1, w1, 0.0)
    w_ref[1:2, :] = jnp.where(kept2, w2, 0.0)


def _run_router(xf, Wr, br, C):
    T, D = xf.shape
    E = Wr.shape[1]
    n = T // _TB
    out1 = jax.ShapeDtypeStruct((2, T), jnp.int32)
    outf = jax.ShapeDtypeStruct((2, T), jnp.float32)
    vec_spec = pl.BlockSpec((2, _TB), lambda i: (0, i))
    return pl.pallas_call(
        functools.partial(_router_body, C, E, _TB),
        grid=(n,),
        in_specs=[
            pl.BlockSpec((_TB, D), lambda i: (i, 0)),
            pl.BlockSpec((D, E), lambda i: (0, 0)),
            pl.BlockSpec((1, E), lambda i: (0, 0)),
        ],
        out_specs=[vec_spec] * 3,
        out_shape=[out1, out1, outf],
        scratch_shapes=[pltpu.VMEM((E, 128), jnp.float32)],
        compiler_params=pltpu.CompilerParams(
            dimension_semantics=("arbitrary",)),
    )(xf, Wr, br.reshape(1, E))


def _run_dispatch(xf, dst, rows_out):
    T, D = xf.shape
    tpw = T // _NW
    mesh = plsc.VectorSubcoreMesh(core_axis_name="c", subcore_axis_name="s")

    @functools.partial(
        pl.kernel,
        out_type=jax.ShapeDtypeStruct((rows_out, D), jnp.float32),
        mesh=mesh,
        scratch_types=[
            pltpu.VMEM((tpw, D), jnp.float32),
            pltpu.VMEM((2, tpw), jnp.int32),
            pltpu.SemaphoreType.DMA,
            pltpu.SemaphoreType.DMA,
        ],
    )
    def disp(xf_hbm, dst_hbm, xe_hbm, rows_v, idx_v, sem1, sem2):
        wid = lax.axis_index("s") * _NC + lax.axis_index("c")
        base = wid * tpw
        pltpu.sync_copy(xf_hbm.at[pl.ds(base, tpw)], rows_v)
        pltpu.sync_copy(dst_hbm.at[:, pl.ds(base, tpw)], idx_v)
        cp1 = pltpu.async_copy(rows_v, xe_hbm.at[idx_v.at[0]], sem1)
        cp2 = pltpu.async_copy(rows_v, xe_hbm.at[idx_v.at[1]], sem2)
        cp1.wait()
        cp2.wait()

    return disp(xf, dst)


_EPB = 2


def _ffn_body(C, x_ref, w1_ref, b1_ref, w2_ref, b2_ref, o_ref):
    for j in range(_EPB):
        xb = x_ref[pl.ds(j * C, C), :]
        h = (jnp.dot(xb, w1_ref[j], preferred_element_type=jnp.float32,
                     precision=lax.Precision.DEFAULT) + b1_ref[j])
        h = 0.5 * h * (1.0 + lax.erf(h * (1.0 / math.sqrt(2.0))))
        o_ref[j] = (jnp.dot(h, w2_ref[j], preferred_element_type=jnp.float32,
                            precision=lax.Precision.DEFAULT) + b2_ref[j])


def _run_ffn(xe, W1, b1, W2, b2, C):
    E, D, F = W1.shape
    return pl.pallas_call(
        functools.partial(_ffn_body, C),
        grid=(E // _EPB,),
        in_specs=[
            pl.BlockSpec((_EPB * C, D), lambda e: (e, 0)),
            pl.BlockSpec((_EPB, D, F), lambda e: (e, 0, 0)),
            pl.BlockSpec((_EPB, 1, F), lambda e: (e, 0, 0)),
            pl.BlockSpec((_EPB, F, D), lambda e: (e, 0, 0)),
            pl.BlockSpec((_EPB, 1, D), lambda e: (e, 0, 0)),
        ],
        out_specs=pl.BlockSpec((_EPB, C, D), lambda e: (e, 0, 0)),
        out_shape=jax.ShapeDtypeStruct((E, C, D), jnp.float32),
        compiler_params=pltpu.CompilerParams(
            dimension_semantics=("arbitrary",)),
    )(xe, W1, b1.reshape(E, 1, F), W2, b2.reshape(E, 1, D))


def _run_combine(ye, cidx, cw, T, D):
    tpw = T // _NW
    nq = 4
    qr = tpw // nq
    nvec = D // 16
    mesh = plsc.VectorSubcoreMesh(core_axis_name="c", subcore_axis_name="s")

    @functools.partial(
        pl.kernel,
        out_type=jax.ShapeDtypeStruct((T, D), jnp.float32),
        mesh=mesh,
        scratch_types=[
            pltpu.VMEM((qr, D), jnp.float32),
            pltpu.VMEM((qr, D), jnp.float32),
            pltpu.VMEM((qr, D), jnp.float32),
            pltpu.VMEM((qr, D), jnp.float32),
            pltpu.VMEM((2, tpw), jnp.int32),
            pltpu.VMEM((tpw,), jnp.float32),
            pltpu.VMEM((tpw,), jnp.float32),
            pltpu.SemaphoreType.DMA,
            pltpu.SemaphoreType.DMA,
            pltpu.SemaphoreType.DMA,
        ],
        compiler_params=pltpu.CompilerParams(needs_layout_passes=False),
    )
    def comb(ye_hbm, ci_hbm, cw_hbm, out_hbm, ga1, ga2, gb1, gb2, idx_v,
             w1_v, w2_v, sema, semb, semo):
        wid = lax.axis_index("s") * _NC + lax.axis_index("c")
        base = wid * tpw
        pltpu.sync_copy(ci_hbm.at[:, pl.ds(base, tpw)], idx_v)
        pltpu.sync_copy(cw_hbm.at[0, pl.ds(base, tpw)], w1_v)
        pltpu.sync_copy(cw_hbm.at[1, pl.ds(base, tpw)], w2_v)

        bufs = [(ga1, ga2, sema), (gb1, gb2, semb)]

        def gather(q):
            g1, g2, sem = bufs[q % 2]
            c1 = pltpu.async_copy(
                ye_hbm.at[idx_v.at[0, pl.ds(q * qr, qr)]], g1, sem)
            c2 = pltpu.async_copy(
                ye_hbm.at[idx_v.at[1, pl.ds(q * qr, qr)]], g2, sem)
            return c1, c2

        pend = {0: gather(0)}
        owrites = []
        for q in range(nq):
            g1, g2, sem = bufs[q % 2]
            if q + 1 < nq:
                if q - 1 >= 0:
                    owrites[q - 1].wait()
                pend[q + 1] = gather(q + 1)
            c1, c2 = pend.pop(q)
            c1.wait()
            c2.wait()

            def row(r, _, q=q, g1=g1, g2=g2):
                ridx = jnp.full((16,), q * qr + r, jnp.int32)
                w1s = plsc.load_gather(w1_v, [ridx])
                w2s = plsc.load_gather(w2_v, [ridx])
                zero = jnp.zeros((16,), jnp.float32)
                for j in range(nvec):
                    a = g1[r, pl.ds(j * 16, 16)]
                    b = g2[r, pl.ds(j * 16, 16)]
                    va = jnp.where(w1s != 0.0, a * w1s, zero)
                    vb = jnp.where(w2s != 0.0, b * w2s, zero)
                    g1[r, pl.ds(j * 16, 16)] = va + vb
                return 0

            lax.fori_loop(0, qr, row, 0)
            owrites.append(pltpu.async_copy(
                g1, out_hbm.at[pl.ds(base + q * qr, qr)], semo))
        owrites[-2].wait()
        owrites[-1].wait()

    return comb(ye, cidx, cw)


def kernel(x, Wr, br, W1, b1, W2, b2):
    B, S, D = x.shape
    T = B * S
    E = Wr.shape[1]
    F = W1.shape[2]
    C = max(1, int(math.ceil(_CF * T * _TOPK / E)))
    EC = E * C
    rows_out = EC + 2 * _NW

    xf = x.reshape(T, D)
    dst, cidx, cw = _run_router(xf, Wr, br, C)
    xe = _run_dispatch(xf, dst, rows_out)
    ye = _run_ffn(xe, W1, b1, W2, b2, C)
    out = _run_combine(ye.reshape(EC, D), cidx, cw, T, D)
    return out.reshape(B, S, D)
